# baseline (speedup 1.0000x reference)
import jax
import jax.numpy as jnp
from jax import lax
from jax.experimental import pallas as pl
from jax.experimental.pallas import tpu as pltpu, tpu_sc as plsc

NUM_FACTORS = 16
BATCH = 16384
NC, NS = 2, 16
NW = NC * NS
BPW = BATCH // NW
CHUNK = 128
NCHUNK = BPW // CHUNK


def _sc_body(users_hbm, movies_hbm, uf_hbm, mf_hbm, out_hbm,
             uidx, midx, urows, mrows, outv, sem):
    wid = lax.axis_index("s") * NC + lax.axis_index("c")
    base = wid * BPW
    pltpu.sync_copy(users_hbm.at[pl.ds(base, BPW)], uidx)
    pltpu.sync_copy(movies_hbm.at[pl.ds(base, BPW)], midx)
    copies = []
    for k in range(NCHUNK):
        sl = pl.ds(k * CHUNK, CHUNK)
        copies.append(pltpu.async_copy(
            uf_hbm.at[uidx.at[sl]], urows.at[sl, :], sem))
        copies.append(pltpu.async_copy(
            mf_hbm.at[midx.at[sl]], mrows.at[sl, :], sem))
    for c in copies:
        c.wait()

    iota16 = lax.broadcasted_iota(jnp.int32, (NUM_FACTORS,), 0)
    dn = lax.GatherDimensionNumbers(
        offset_dims=(), collapsed_slice_dims=(0,), start_index_map=(0,))

    def perm(v, k):
        return lax.gather(v, (iota16 ^ k)[:, None], dn, slice_sizes=(1,),
                          mode=lax.GatherScatterMode.PROMISE_IN_BOUNDS)

    def group_body(g, carry):
        base_row = g * NUM_FACTORS
        acc = jnp.zeros((NUM_FACTORS,), jnp.float32)
        for j in range(NUM_FACTORS):
            i = base_row + j
            t = urows[i, :] * mrows[i, :]
            for k in (1, 2, 4, 8):
                t = t + perm(t, k)
            acc = jnp.where(iota16 == j, t, acc)
        outv[pl.ds(base_row, NUM_FACTORS)] = acc
        return carry

    lax.fori_loop(0, BPW // NUM_FACTORS, group_body, 0)
    pltpu.sync_copy(outv, out_hbm.at[pl.ds(base, BPW)])


def kernel(data, user_factors, movie_factors):
    users = data[:, 0].astype(jnp.int32)
    movies = data[:, 1].astype(jnp.int32)
    mesh = plsc.VectorSubcoreMesh(core_axis_name="c", subcore_axis_name="s",
                                  num_cores=NC, num_subcores=NS)
    f = pl.kernel(
        _sc_body,
        out_type=jax.ShapeDtypeStruct((BATCH,), jnp.float32),
        mesh=mesh,
        scratch_types=[
            pltpu.VMEM((BPW,), jnp.int32),
            pltpu.VMEM((BPW,), jnp.int32),
            pltpu.VMEM((BPW, NUM_FACTORS), jnp.float32),
            pltpu.VMEM((BPW, NUM_FACTORS), jnp.float32),
            pltpu.VMEM((BPW,), jnp.float32),
            pltpu.SemaphoreType.DMA,
        ],
        compiler_params=pltpu.CompilerParams(use_tc_tiling_on_sc=False),
    )
    return f(users, movies, user_factors, movie_factors)
